# baseline (speedup 1.0000x reference)
import functools

import jax
import jax.numpy as jnp
import numpy as np
from jax import lax
from jax.experimental import pallas as pl
from jax.experimental.pallas import tpu as pltpu
from jax.experimental.pallas import tpu_sc as plsc

NC = 2
NS = 16
NW = NC * NS

GW = 128
K = 8
CH = K * GW
NBUF = 2


@functools.partial(jax.jit, static_argnums=(2, 3))
def _embedding_gather(tokens_flat, table_bf, b_per_w, n_chunks):
    B = tokens_flat.shape[0]
    D = table_bf.shape[1]
    H = D // 2

    mesh = plsc.VectorSubcoreMesh(core_axis_name="c", subcore_axis_name="s")

    @functools.partial(
        pl.kernel,
        out_type=jax.ShapeDtypeStruct((B, D), jnp.float32),
        mesh=mesh,
        scratch_types=[
            pltpu.VMEM((NBUF, CH), jnp.int32),
            pltpu.VMEM((NBUF, CH, D), jnp.bfloat16),
            pltpu.VMEM((NBUF, CH, D), jnp.float32),
            pltpu.SemaphoreType.DMA((NBUF,)),
            pltpu.SemaphoreType.DMA((NBUF,)),
            pltpu.SemaphoreType.DMA((NBUF,)),
        ],
        compiler_params=pltpu.CompilerParams(
            use_tc_tiling_on_sc=False, needs_layout_passes=False
        ),
    )
    def k(tok_hbm, table_hbm, out_hbm, idx_v, rows_bf, rows_f, sem_i, sem_g,
          sem_o):
        wid = lax.axis_index("s") * NC + lax.axis_index("c")
        base = wid * b_per_w

        def idx_copy(c, b):
            return pltpu.make_async_copy(
                tok_hbm.at[pl.ds(base + c * CH, CH)], idx_v.at[b], sem_i.at[b]
            )

        def out_copy(c, b):
            return pltpu.make_async_copy(
                rows_f.at[b], out_hbm.at[pl.ds(base + c * CH, CH)], sem_o.at[b]
            )

        for b in range(NBUF):
            idx_copy(b, b).start()

        def body(it, carry):
            for b in range(NBUF):
                c = it * NBUF + b
                idx_copy(c, b).wait()

                gathers = [
                    pltpu.async_copy(
                        table_hbm.at[idx_v.at[b].at[pl.ds(j * GW, GW)]],
                        rows_bf.at[b].at[pl.ds(j * GW, GW)],
                        sem_g.at[b],
                    )
                    for j in range(K)
                ]
                for g in gathers:
                    g.wait()

                @pl.when(it > 0)
                def _():
                    out_copy(c - NBUF, b).wait()

                @pl.when(c + NBUF < n_chunks)
                def _():
                    idx_copy(c + NBUF, b).start()

                def conv(i, carry2):
                    x = rows_bf.at[b][i, :]
                    lo, hi = plsc.unpack(
                        x,
                        format=plsc.PackFormat.INTERLEAVED,
                        preferred_element_type=jnp.float32,
                    )
                    rows_f.at[b][i, pl.ds(0, H)] = lo
                    rows_f.at[b][i, pl.ds(H, H)] = hi
                    return carry2

                lax.fori_loop(0, CH, conv, 0, unroll=16)

                out_copy(c, b).start()

            return carry

        lax.fori_loop(0, n_chunks // NBUF, body, 0)

        for b in range(NBUF):
            out_copy(n_chunks - NBUF + b, b).wait()

    return k(tokens_flat, table_bf)


def kernel(tokens, table):
    B0, S = tokens.shape
    V, D = table.shape
    B = B0 * S
    b_per_w = B // NW
    n_chunks = b_per_w // CH
    assert B % NW == 0 and b_per_w % (CH * NBUF) == 0

    H = D // 2
    perm = np.empty(D, dtype=np.int32)
    perm[0::2] = np.arange(H)
    perm[1::2] = np.arange(H, D)
    table_bf = table[:, perm].astype(jnp.bfloat16)

    out = _embedding_gather(tokens.reshape(B), table_bf, b_per_w, n_chunks)
    return out.reshape(B0, S, D)


# speedup vs baseline: 3.3436x; 3.3436x over previous
import functools

import jax
import jax.numpy as jnp
from jax import lax
from jax.experimental import pallas as pl
from jax.experimental.pallas import tpu as pltpu
from jax.experimental.pallas import tpu_sc as plsc

NC = 2
NS = 16
NW = NC * NS

GW = 128
K = 8
CH = K * GW
NBUF = 2

TCW = 8192
TCR = 128


@functools.partial(jax.jit, static_argnums=(2, 3))
def _embedding_gather_bf16(tokens_flat, table_bf, b_per_w, n_chunks):
    B = tokens_flat.shape[0]
    D = table_bf.shape[1]

    mesh = plsc.VectorSubcoreMesh(core_axis_name="c", subcore_axis_name="s")

    @functools.partial(
        pl.kernel,
        out_type=jax.ShapeDtypeStruct((B, D), jnp.bfloat16),
        mesh=mesh,
        scratch_types=[
            pltpu.VMEM((NBUF, CH), jnp.int32),
            pltpu.VMEM((NBUF, CH, D), jnp.bfloat16),
            pltpu.SemaphoreType.DMA((NBUF,)),
            pltpu.SemaphoreType.DMA((NBUF,)),
            pltpu.SemaphoreType.DMA((NBUF,)),
        ],
        compiler_params=pltpu.CompilerParams(use_tc_tiling_on_sc=False),
    )
    def k(tok_hbm, table_hbm, out_hbm, idx_v, rows_v, sem_i, sem_g, sem_o):
        wid = lax.axis_index("s") * NC + lax.axis_index("c")
        base = wid * b_per_w

        def idx_copy(c, b):
            return pltpu.make_async_copy(
                tok_hbm.at[pl.ds(base + c * CH, CH)], idx_v.at[b], sem_i.at[b]
            )

        def out_copy(c, b):
            return pltpu.make_async_copy(
                rows_v.at[b], out_hbm.at[pl.ds(base + c * CH, CH)], sem_o.at[b]
            )

        for b in range(NBUF):
            idx_copy(b, b).start()

        def body(it, carry):
            for b in range(NBUF):
                c = it * NBUF + b
                idx_copy(c, b).wait()

                @pl.when(it > 0)
                def _():
                    out_copy(c - NBUF, b).wait()

                gathers = [
                    pltpu.async_copy(
                        table_hbm.at[idx_v.at[b].at[pl.ds(j * GW, GW)]],
                        rows_v.at[b].at[pl.ds(j * GW, GW)],
                        sem_g.at[b],
                    )
                    for j in range(K)
                ]
                for g in gathers:
                    g.wait()

                out_copy(c, b).start()

                @pl.when(c + NBUF < n_chunks)
                def _():
                    idx_copy(c + NBUF, b).start()

            return carry

        lax.fori_loop(0, n_chunks // NBUF, body, 0)

        for b in range(NBUF):
            out_copy(n_chunks - NBUF + b, b).wait()

    return k(tokens_flat, table_bf)


def _upcast_block(i_ref, o_ref):
    o_ref[...] = i_ref[...].astype(jnp.float32)


@jax.jit
def _upcast_f32(x_bf):
    R = x_bf.shape[0]
    grid = (R // TCR,)
    return pl.pallas_call(
        _upcast_block,
        grid=grid,
        in_specs=[pl.BlockSpec((TCR, TCW), lambda i: (i, 0))],
        out_specs=pl.BlockSpec((TCR, TCW), lambda i: (i, 0)),
        out_shape=jax.ShapeDtypeStruct((R, TCW), jnp.float32),
    )(x_bf)


def kernel(tokens, table):
    B0, S = tokens.shape
    V, D = table.shape
    B = B0 * S
    b_per_w = B // NW
    n_chunks = b_per_w // CH
    assert B % NW == 0 and b_per_w % (CH * NBUF) == 0
    assert (B * D) % (TCR * TCW) == 0

    table_bf = table.astype(jnp.bfloat16)
    out_bf = _embedding_gather_bf16(tokens.reshape(B), table_bf, b_per_w,
                                    n_chunks)
    out = _upcast_f32(out_bf.reshape(B * D // TCW, TCW))
    return out.reshape(B0, S, D)


# speedup vs baseline: 5.4107x; 1.6182x over previous
import functools

import jax
import jax.numpy as jnp
from jax import lax
from jax.experimental import pallas as pl
from jax.experimental.pallas import tpu as pltpu
from jax.experimental.pallas import tpu_sc as plsc

NC = 2
NS = 16
NW = NC * NS

GW = 128
K = 8
CH = K * GW
NBUF = 2


@functools.partial(jax.jit, static_argnums=(2, 3))
def _embedding_gather(tokens_flat, table, b_per_w, n_chunks):
    B = tokens_flat.shape[0]
    D = table.shape[1]
    H = D // 2

    mesh = plsc.VectorSubcoreMesh(core_axis_name="c", subcore_axis_name="s")

    @functools.partial(
        pl.kernel,
        out_type=jax.ShapeDtypeStruct((B * D,), jnp.float32),
        mesh=mesh,
        scratch_types=[
            pltpu.VMEM((NBUF, CH), jnp.int32),
            pltpu.VMEM((NBUF, CH, D), jnp.float32),
            pltpu.VMEM((CH * D,), jnp.float32),
            pltpu.SemaphoreType.DMA((NBUF,)),
            pltpu.SemaphoreType.DMA((NBUF,)),
            pltpu.SemaphoreType.DMA((NBUF,)),
        ],
        compiler_params=pltpu.CompilerParams(
            use_tc_tiling_on_sc=False, needs_layout_passes=False
        ),
    )
    def k(tok_hbm, table_hbm, out_hbm, idx_v, rows_v, stage_v, sem_i, sem_g,
          sem_o):
        wid = lax.axis_index("s") * NC + lax.axis_index("c")
        base = wid * b_per_w
        iota = lax.iota(jnp.int32, 16)

        def idx_copy(c, b):
            return pltpu.make_async_copy(
                tok_hbm.at[pl.ds(base + c * CH, CH)], idx_v.at[b], sem_i.at[b]
            )

        def out_copy(c, b):
            return pltpu.make_async_copy(
                stage_v, out_hbm.at[pl.ds((base + c * CH) * D, CH * D)],
                sem_o.at[b],
            )

        for b in range(NBUF):
            idx_copy(b, b).start()

        def body(it, carry):
            for b in range(NBUF):
                c = it * NBUF + b
                idx_copy(c, b).wait()

                gathers = [
                    pltpu.async_copy(
                        table_hbm.at[idx_v.at[b].at[pl.ds(j * GW, GW)]],
                        rows_v.at[b].at[pl.ds(j * GW, GW)],
                        sem_g.at[b],
                    )
                    for j in range(K)
                ]
                for g in gathers:
                    g.wait()

                @pl.when(c > 0)
                def _():
                    out_copy(c - 1, 1 - b).wait()

                @functools.partial(plsc.parallel_loop, 0, CH, unroll=4)
                def place(r):
                    lo = rows_v.at[b][r, pl.ds(0, H)]
                    hi = rows_v.at[b][r, pl.ds(H, H)]
                    tgt = r * D + iota
                    plsc.store_scatter(stage_v, [tgt], lo)
                    plsc.store_scatter(stage_v, [tgt + H], hi)

                out_copy(c, b).start()

                @pl.when(c + NBUF < n_chunks)
                def _():
                    idx_copy(c + NBUF, b).start()

            return carry

        lax.fori_loop(0, n_chunks // NBUF, body, 0)

        out_copy(n_chunks - 1, (n_chunks - 1) % NBUF).wait()

    return k(tokens_flat, table)


def kernel(tokens, table):
    B0, S = tokens.shape
    V, D = table.shape
    B = B0 * S
    b_per_w = B // NW
    n_chunks = b_per_w // CH
    assert B % NW == 0 and b_per_w % (CH * NBUF) == 0

    out = _embedding_gather(tokens.reshape(B), table, b_per_w, n_chunks)
    return out.reshape(B0, S, D)
